# Initial kernel scaffold; baseline (speedup 1.0000x reference)
import jax
import jax.numpy as jnp
from jax.experimental import pallas as pl


def kernel(vector_embeddings, scalar_embeddings, edge_index, edge_vectors, W_filter, b_filter, W1, b1, W2, b2):
    raise NotImplementedError("write your pallas kernel here")


# speedup vs baseline: 28.6343x; 28.6343x over previous
import functools

import jax
import jax.numpy as jnp
from jax import lax
from jax.experimental import pallas as pl
from jax.experimental.pallas import tpu as pltpu
from jax.experimental.pallas import tpu_sc as plsc

RADIAL = 20
DIM = 128
CUTOFF = 5.0
N_NODES = 10000
N_EDGES = 320000

TPAD = 32
TW = 4 * TPAD

EB = 1600
NBK = 400

NC = 2
NS = 16
NW = NC * NS
EPW = N_EDGES // NW
CH = 80
NCH = EPW // CH
RPS = 640
RPS_LAST = N_NODES - (NS - 1) * RPS


def _feat_body(ev_ref, t_ref):
    ev = ev_ref[...]
    x = ev[:, 0:1]
    y = ev[:, 1:2]
    z = ev[:, 2:3]
    d2 = x * x + y * y + z * z
    d = jnp.sqrt(d2)
    inv = 1.0 / d
    r = d * (1.0 / CUTOFF)
    r2 = r * r
    r4 = r2 * r2
    r6 = r4 * r2
    r7 = r6 * r
    r8 = r7 * r
    env = jnp.where(d < CUTOFF, 1.0 - 28.0 * r6 + 48.0 * r7 - 21.0 * r8, 0.0)
    nvec = (lax.broadcasted_iota(jnp.int32, (1, TPAD), 1) + 1).astype(jnp.float32)
    s = jnp.sin(d * (jnp.pi / CUTOFF) * nvec)
    bes = s * (inv * jnp.sqrt(2.0 / CUTOFF))
    col = lax.broadcasted_iota(jnp.int32, (1, TPAD), 1)
    ge = bes * env
    g = jnp.where(col < RADIAL, ge,
                  jnp.where(col == RADIAL, jnp.broadcast_to(env, ge.shape), 0.0))
    t = jnp.concatenate([g, g * (x * inv), g * (y * inv), g * (z * inv)], axis=1)
    t_ref[...] = t


def _featurize(edge_vectors):
    return pl.pallas_call(
        _feat_body,
        grid=(N_EDGES // EB,),
        in_specs=[pl.BlockSpec((EB, 3), lambda i: (i, 0))],
        out_specs=pl.BlockSpec((EB, TW), lambda i: (i, 0)),
        out_shape=jax.ShapeDtypeStruct((N_EDGES, TW), jnp.float32),
    )(edge_vectors)


def _scatter_body(t_hbm, idx_hbm, zeros_hbm, out_hbm, idx_v, rows_v, acc_sh):
    cid = lax.axis_index("c")
    sid = lax.axis_index("s")
    wid = sid * NC + cid
    @pl.when(sid < NS - 1)
    def _():
        pltpu.sync_copy(zeros_hbm.at[pl.ds(sid * RPS, RPS)],
                        acc_sh.at[pl.ds(sid * RPS, RPS)])

    @pl.when(sid == NS - 1)
    def _():
        pltpu.sync_copy(zeros_hbm.at[pl.ds((NS - 1) * RPS, RPS_LAST)],
                        acc_sh.at[pl.ds((NS - 1) * RPS, RPS_LAST)])

    plsc.subcore_barrier()
    base = wid * EPW

    @pl.loop(0, NCH)
    def _(k):
        off = pl.multiple_of(base + k * CH, CH)
        pltpu.sync_copy(idx_hbm.at[pl.ds(off, CH)], idx_v)
        pltpu.sync_copy(t_hbm.at[pl.ds(off, CH)], rows_v)
        pltpu.sync_copy(rows_v, acc_sh.at[idx_v], add=True)

    plsc.subcore_barrier()

    @pl.when(sid < NS - 1)
    def _():
        pltpu.sync_copy(acc_sh.at[pl.ds(sid * RPS, RPS)],
                        out_hbm.at[cid, pl.ds(sid * RPS, RPS)])

    @pl.when(sid == NS - 1)
    def _():
        pltpu.sync_copy(acc_sh.at[pl.ds((NS - 1) * RPS, RPS_LAST)],
                        out_hbm.at[cid, pl.ds((NS - 1) * RPS, RPS_LAST)])


def _scatter_sc(t, nbr, zeros):
    mesh = plsc.VectorSubcoreMesh(core_axis_name="c", subcore_axis_name="s")
    f = functools.partial(
        pl.kernel,
        out_type=jax.ShapeDtypeStruct((NC, N_NODES, TW), jnp.float32),
        mesh=mesh,
        scratch_types=[
            pltpu.VMEM((CH,), jnp.int32),
            pltpu.VMEM((CH, TW), jnp.float32),
            pltpu.VMEM_SHARED((N_NODES, TW), jnp.float32),
        ],
    )(_scatter_body)
    return f(t, nbr, zeros)


def _combine_body(p_ref, se_ref, vt_ref, wf_ref, w1_ref, b1_ref, w2_ref,
                  b2_ref, dv_ref, ds_ref):
    p = p_ref[...]
    T = p[0] + p[1]
    se = se_ref[...]
    hp = lax.dot(se, w1_ref[...], precision=lax.Precision.HIGHEST,
                 preferred_element_type=jnp.float32) + b1_ref[...]
    h = hp * (1.0 / (1.0 + jnp.exp(-hp)))
    phi = lax.dot(h, w2_ref[...], precision=lax.Precision.HIGHEST,
                  preferred_element_type=jnp.float32) + b2_ref[...]
    wf = wf_ref[...]
    G0 = T[:, 0:TPAD]
    Sa = lax.dot(G0, wf[:, 0:DIM], precision=lax.Precision.HIGHEST,
                 preferred_element_type=jnp.float32)
    Sc = lax.dot(G0, wf[:, 2 * DIM:3 * DIM], precision=lax.Precision.HIGHEST,
                 preferred_element_type=jnp.float32)
    ds = phi[:, 0:DIM] * Sa
    pc = phi[:, 2 * DIM:3 * DIM] * Sc
    phib = phi[:, DIM:2 * DIM]
    vt = vt_ref[...]
    dvs = []
    for j in range(3):
        Gj = T[:, TPAD * (j + 1):TPAD * (j + 2)]
        Sbj = lax.dot(Gj, wf[:, DIM:2 * DIM], precision=lax.Precision.HIGHEST,
                      preferred_element_type=jnp.float32)
        dvs.append(phib * Sbj + pc * vt[j])
    dv_ref[...] = jnp.stack(dvs, axis=0)
    ds_ref[...] = ds


def _combine(P, se, vt, Wf, W1, b1, W2, b2):
    grid = (N_NODES // NBK,)
    dv, ds = pl.pallas_call(
        _combine_body,
        grid=grid,
        in_specs=[
            pl.BlockSpec((NC, NBK, TW), lambda i: (0, i, 0)),
            pl.BlockSpec((NBK, DIM), lambda i: (i, 0)),
            pl.BlockSpec((3, NBK, DIM), lambda i: (0, i, 0)),
            pl.BlockSpec((TPAD, 3 * DIM), lambda i: (0, 0)),
            pl.BlockSpec((DIM, DIM), lambda i: (0, 0)),
            pl.BlockSpec((1, DIM), lambda i: (0, 0)),
            pl.BlockSpec((DIM, 3 * DIM), lambda i: (0, 0)),
            pl.BlockSpec((1, 3 * DIM), lambda i: (0, 0)),
        ],
        out_specs=[
            pl.BlockSpec((3, NBK, DIM), lambda i: (0, i, 0)),
            pl.BlockSpec((NBK, DIM), lambda i: (i, 0)),
        ],
        out_shape=[
            jax.ShapeDtypeStruct((3, N_NODES, DIM), jnp.float32),
            jax.ShapeDtypeStruct((N_NODES, DIM), jnp.float32),
        ],
    )(P, se, vt, Wf, W1, b1, W2, b2)
    return dv, ds


def kernel(vector_embeddings, scalar_embeddings, edge_index, edge_vectors,
           W_filter, b_filter, W1, b1, W2, b2):
    nbr = edge_index[1]
    t = _featurize(edge_vectors)
    zeros = jnp.zeros((N_NODES, TW), jnp.float32)
    P = _scatter_sc(t, nbr, zeros)
    Wf = jnp.concatenate(
        [W_filter, b_filter[None, :], jnp.zeros((TPAD - RADIAL - 1, 3 * DIM),
                                                jnp.float32)], axis=0)
    vt = jnp.transpose(vector_embeddings, (2, 0, 1))
    dv, ds = _combine(P, scalar_embeddings, vt, Wf, W1,
                      b1[None, :], W2, b2[None, :])
    return jnp.transpose(dv, (1, 2, 0)), ds
